# baseline (speedup 1.0000x reference)
import functools

import jax
import jax.numpy as jnp
from jax import lax
from jax.experimental import pallas as pl
from jax.experimental.pallas import tpu as pltpu
from jax.experimental.pallas import tpu_sc as plsc

B = 16384
NC, NS, L = 1, 16, 16
NW = NC * NS
BPW = B // NW

_mesh = plsc.VectorSubcoreMesh(core_axis_name="c", subcore_axis_name="s",
                               num_cores=1)


@functools.partial(
    pl.kernel,
    mesh=_mesh,
    compiler_params=pltpu.CompilerParams(needs_layout_passes=False),
    out_type=jax.ShapeDtypeStruct((2, B), jnp.float32),
    scratch_types=[
        pltpu.VMEM((L,), jnp.float32),
        pltpu.VMEM((2, 2), jnp.float32),
        pltpu.VMEM((BPW,), jnp.int32),
        pltpu.VMEM((2 * BPW,), jnp.float32)
    ],
)
def _emission_sc(logits_hbm, x_hbm, out_hbm, t_v, l_v, x_v, o_v):
    wid = lax.axis_index("s") * NC + lax.axis_index("c")
    base = wid * BPW
    pltpu.sync_copy(x_hbm.at[pl.ds(base, BPW)], x_v)
    pltpu.sync_copy(logits_hbm, l_v)

    lanes = lax.iota(jnp.int32, L)
    partner = lanes ^ 1
    v = plsc.load_gather(l_v, [(lanes >> 1) & 1, lanes & 1])
    t_v[...] = v
    pv = plsc.load_gather(t_v, [partner])
    e = jnp.exp(v - jnp.maximum(v, pv))
    t_v[...] = e
    ep = plsc.load_gather(t_v, [partner])
    p = e / (e + ep)
    t_v[...] = p
    one = lanes * 0 + 1
    p00 = plsc.load_gather(t_v, [one + 7])
    p01 = plsc.load_gather(t_v, [one + 8])
    p10 = plsc.load_gather(t_v, [one + 9])
    p11 = plsc.load_gather(t_v, [one + 10])

    def body(i, carry):
        xv = plsc.load_gather(x_v, [lanes + i * L])
        msk = xv == 0
        o_v[pl.ds(i * L, L)] = jnp.where(msk, p00, p01)
        o_v[pl.ds(BPW + i * L, L)] = jnp.where(msk, p10, p11)
        return carry

    lax.fori_loop(0, BPW // L, body, 0)
    pltpu.sync_copy(o_v.at[pl.ds(0, BPW)], out_hbm.at[0, pl.ds(base, BPW)])
    pltpu.sync_copy(o_v.at[pl.ds(BPW, BPW)], out_hbm.at[1, pl.ds(base, BPW)])


def kernel(x_t, emission_logits):
    return _emission_sc(emission_logits.astype(jnp.float32),
                        x_t.astype(jnp.int32)).T


# speedup vs baseline: 1.0222x; 1.0222x over previous
import functools

import jax
import jax.numpy as jnp
from jax import lax
from jax.experimental import pallas as pl
from jax.experimental.pallas import tpu as pltpu
from jax.experimental.pallas import tpu_sc as plsc

B = 16384
NC, NS, L = 1, 16, 16
NW = NC * NS
BPW = B // NW

_mesh = plsc.VectorSubcoreMesh(core_axis_name="c", subcore_axis_name="s",
                               num_cores=1)


@functools.partial(
    pl.kernel,
    mesh=_mesh,
    compiler_params=pltpu.CompilerParams(needs_layout_passes=False),
    out_type=jax.ShapeDtypeStruct((2, B), jnp.float32),
    scratch_types=[
        pltpu.VMEM((2 * L,), jnp.float32),
        pltpu.VMEM((BPW,), jnp.int32),
        pltpu.VMEM((2 * BPW,), jnp.float32)
    ],
)
def _emission_sc(logits_hbm, x_hbm, out_hbm, t_v, x_v, o_v):
    wid = lax.axis_index("s") * NC + lax.axis_index("c")
    base = wid * BPW
    pltpu.sync_copy(x_hbm.at[pl.ds(base, BPW)], x_v)
    pltpu.sync_copy(logits_hbm, t_v.at[pl.ds(24, 4)])

    lanes = lax.iota(jnp.int32, L)
    partner = lanes ^ 1
    v = t_v[pl.ds(16, L)]
    t_v[pl.ds(0, L)] = v
    pv = plsc.load_gather(t_v, [partner])
    e = jnp.exp(v - jnp.maximum(v, pv))
    t_v[pl.ds(0, L)] = e
    ep = plsc.load_gather(t_v, [partner])
    p = e / (e + ep)
    t_v[pl.ds(0, L)] = p
    one = lanes * 0 + 1
    p00 = plsc.load_gather(t_v, [one + 7])
    p01 = plsc.load_gather(t_v, [one + 8])
    p10 = plsc.load_gather(t_v, [one + 9])
    p11 = plsc.load_gather(t_v, [one + 10])

    def body(i, carry):
        xv = plsc.load_gather(x_v, [lanes + i * L])
        msk = xv == 0
        o_v[pl.ds(i * L, L)] = jnp.where(msk, p00, p01)
        o_v[pl.ds(BPW + i * L, L)] = jnp.where(msk, p10, p11)
        return carry

    lax.fori_loop(0, BPW // L, body, 0)
    pltpu.sync_copy(o_v.at[pl.ds(0, BPW)], out_hbm.at[0, pl.ds(base, BPW)])
    pltpu.sync_copy(o_v.at[pl.ds(BPW, BPW)], out_hbm.at[1, pl.ds(base, BPW)])


def kernel(x_t, emission_logits):
    logits_flat = emission_logits.reshape(-1).astype(jnp.float32)
    return _emission_sc(logits_flat, x_t.astype(jnp.int32)).T
